# baseline (speedup 1.0000x reference)
import functools

import jax
import jax.numpy as jnp
from jax import lax
from jax.experimental import pallas as pl
from jax.experimental.pallas import tpu as pltpu
from jax.experimental.pallas import tpu_sc as plsc

N = 2_000_000
C = 4_000
NCHUNK = N // C
NT = 16
NVEC = C // 16
HROWS = 512
VROWS = 256
OPT = 0.05
R = 32

_I32MIN = -2147483648


def _sc_body(xyz_hbm, scl_hbm, op_hbm, scal_out, grid_out,
             hist_v, xyzbuf, sclbuf, opbuf, histslice, accbuf, resbuf,
             outbuf, idxrows, allacc, hist_sh, acc_sh):
    wid = lax.axis_index("s")
    IOTA = lax.broadcasted_iota(jnp.int32, (16,), 0)
    ZERO16 = jnp.zeros((16,), jnp.float32)
    ONES16 = jnp.full((16,), 1.0, jnp.float32)
    INF16 = jnp.full((16,), jnp.inf, jnp.float32)
    C16 = jnp.full((16,), 16, jnp.int32)
    C31 = jnp.full((16,), 31, jnp.int32)
    C7 = jnp.full((16,), 7, jnp.int32)
    C127 = jnp.full((16,), 127, jnp.int32)
    MINV16 = jnp.full((16,), _I32MIN, jnp.int32)

    def lane_f(v, k):
        return jnp.sum(jnp.where(IOTA == k, v, 0.0))

    def splat_f(s):
        return jnp.full((16,), s, jnp.float32)

    def splat_i(s):
        return jnp.full((16,), s, jnp.int32)

    def zero_hist(nvecs):
        def b(i, _):
            r = lax.shift_right_logical(i, 3)
            c = lax.bitwise_and(i, 7) * 16
            hist_v[r, pl.ds(c, 16)] = ZERO16
            return 0
        lax.fori_loop(0, nvecs, b, 0)

    def combine_hist(nchunks128):
        for k in range(nchunks128):
            pltpu.sync_copy(hist_v.at[pl.ds(k * 128, 128)],
                            hist_sh.at[idxrows.at[k]], add=True)

    def key_of(yv):
        b = plsc.bitcast(yv, jnp.int32)
        m = lax.shift_right_arithmetic(b, C31)
        return lax.bitwise_xor(b, lax.bitwise_or(m, MINV16))

    def dma_chunk(cid, want_scl):
        s3 = pl.multiple_of(cid * 12000, 8)
        s1 = pl.multiple_of(cid * 4000, 8)
        pltpu.sync_copy(xyz_hbm.at[pl.ds(s3, 12000)], xyzbuf.at[pl.ds(0, 12000)])
        if want_scl:
            pltpu.sync_copy(scl_hbm.at[pl.ds(s3, 12000)], sclbuf.at[pl.ds(0, 12000)])
        pltpu.sync_copy(op_hbm.at[pl.ds(s1, 4000)], opbuf.at[pl.ds(0, 4000)])

    nchunks = jnp.where(wid < 4, 32, 31)

    for k in range(4):
        for j in range(8):
            idxrows[k, pl.ds(j * 16, 16)] = IOTA + (k * 128 + j * 16)

    zero_hist(HROWS * 8)

    @pl.when(wid == 0)
    def _():
        pltpu.sync_copy(hist_v, hist_sh)
    plsc.subcore_barrier()

    def vecA(i, carry):
        (mnx, mxx, mny, mxy, mnz, mxz, cnt, S, Sh, Sh2) = carry
        xi = i * 48 + IOTA * 3
        xv = plsc.load_gather(xyzbuf, [xi])
        yv = plsc.load_gather(xyzbuf, [xi + 1])
        zv = plsc.load_gather(xyzbuf, [xi + 2])
        opv = opbuf[pl.ds(i * 16, 16)]
        vis = opv > OPT
        w = jnp.where(vis, opv, 0.0)
        mnx = jnp.minimum(mnx, jnp.where(vis, xv, INF16))
        mxx = jnp.maximum(mxx, jnp.where(vis, xv, -INF16))
        mny = jnp.minimum(mny, jnp.where(vis, yv, INF16))
        mxy = jnp.maximum(mxy, jnp.where(vis, yv, -INF16))
        mnz = jnp.minimum(mnz, jnp.where(vis, zv, INF16))
        mxz = jnp.maximum(mxz, jnp.where(vis, zv, -INF16))
        cnt = cnt + jnp.where(vis, 1.0, 0.0)
        S = S + w
        Sh = Sh + w * yv
        Sh2 = Sh2 + w * yv * yv
        top = lax.shift_right_logical(key_of(yv), C16)
        plsc.addupdate_scatter(
            hist_v,
            [lax.shift_right_logical(top, C7), lax.bitwise_and(top, C127)],
            ONES16, mask=vis)
        return (mnx, mxx, mny, mxy, mnz, mxz, cnt, S, Sh, Sh2)

    def chunkA(kk, carry):
        dma_chunk(wid + kk * 16, False)
        return lax.fori_loop(0, NVEC, vecA, carry)

    accA = lax.fori_loop(0, nchunks, chunkA, (
        INF16, -INF16, INF16, -INF16, INF16, -INF16,
        ZERO16, ZERO16, ZERO16, ZERO16))

    combine_hist(4)
    for j in range(10):
        accbuf[j, :] = accA[j]
    pltpu.sync_copy(accbuf, acc_sh.at[wid])
    plsc.subcore_barrier()
    pltpu.sync_copy(acc_sh, allacc)

    mnx_g, mxx_g = jnp.inf, -jnp.inf
    mny_g, mxy_g = jnp.inf, -jnp.inf
    mnz_g, mxz_g = jnp.inf, -jnp.inf
    n_g, S_g, Sh_g, Sh2_g = 0.0, 0.0, 0.0, 0.0
    for t in range(16):
        mnx_g = jnp.minimum(mnx_g, jnp.min(allacc[t, 0]))
        mxx_g = jnp.maximum(mxx_g, jnp.max(allacc[t, 1]))
        mny_g = jnp.minimum(mny_g, jnp.min(allacc[t, 2]))
        mxy_g = jnp.maximum(mxy_g, jnp.max(allacc[t, 3]))
        mnz_g = jnp.minimum(mnz_g, jnp.min(allacc[t, 4]))
        mxz_g = jnp.maximum(mxz_g, jnp.max(allacc[t, 5]))
        n_g = n_g + jnp.sum(allacc[t, 6])
        S_g = S_g + jnp.sum(allacc[t, 7])
        Sh_g = Sh_g + jnp.sum(allacc[t, 8])
        Sh2_g = Sh2_g + jnp.sum(allacc[t, 9])

    pos = jnp.float32(OPT) * (n_g - 1.0)
    low_i = pos.astype(jnp.int32)
    low_f = low_i.astype(jnp.float32)
    high_f = jnp.where(pos > low_f, low_f + 1.0, low_f)
    hw = pos - low_f
    lw = 1.0 - hw

    def slice_totals():
        pltpu.sync_copy(hist_sh.at[pl.ds(wid * 32, 32)], histslice)

        def b(p, a):
            r = lax.shift_right_logical(p, 3)
            c = lax.bitwise_and(p, 7) * 16
            return a + histslice[r, pl.ds(c, 16)]
        v = lax.fori_loop(0, 256, b, ZERO16)
        accbuf[0, :] = splat_f(jnp.sum(v))
        pltpu.sync_copy(accbuf.at[0], acc_sh.at[wid, 10])

    def find_owner(rank_f):
        cum, owner, below = 0.0, jnp.int32(-1), 0.0
        for t in range(16):
            tot = lane_f(allacc[t, 10], 0)
            ncum = cum + tot
            hit = (ncum >= rank_f + 1.0) & (owner < 0)
            owner = jnp.where(hit, t, owner)
            below = jnp.where(hit, cum, below)
            cum = ncum
        return owner, below

    def owner_scan(rank_f, below_slice, res_row):
        def b(p, car):
            cum, fb, fbel = car
            r = lax.shift_right_logical(p, 3)
            c = lax.bitwise_and(p, 7) * 16
            vec = histslice[r, pl.ds(c, 16)]
            ic = plsc.cumsum(vec) + cum
            hit = ic >= rank_f + 1.0
            any_hit = jnp.any(hit)
            f = jnp.min(plsc.all_reduce_ffs(hit))
            binv = jnp.sum(jnp.where(IOTA == f, vec, 0.0))
            icf = jnp.sum(jnp.where(IOTA == f, ic, 0.0))
            gb = (wid * 4096 + p * 16 + f).astype(jnp.float32)
            upd = any_hit & (fb < 0.0)
            fb = jnp.where(upd, gb, fb)
            fbel = jnp.where(upd, icf - binv, fbel)
            return (cum + jnp.sum(vec), fb, fbel)
        _, fb, fbel = lax.fori_loop(0, 256, b, (below_slice, -1.0, 0.0))
        resbuf[...] = jnp.where(IOTA == 0, splat_f(fb), splat_f(fbel))
        pltpu.sync_copy(resbuf, acc_sh.at[res_row, 11])

    def rank_search(rank_lo_f, rank_hi_f, row_lo, row_hi):
        slice_totals()
        plsc.subcore_barrier()
        pltpu.sync_copy(acc_sh, allacc)
        own_lo, bel_lo = find_owner(rank_lo_f)
        own_hi, bel_hi = find_owner(rank_hi_f)

        @pl.when(wid == own_lo)
        def _():
            owner_scan(rank_lo_f, bel_lo, row_lo)

        @pl.when(wid == own_hi)
        def _():
            owner_scan(rank_hi_f, bel_hi, row_hi)
        plsc.subcore_barrier()
        pltpu.sync_copy(acc_sh, allacc)
        return (lane_f(allacc[row_lo, 11], 0), lane_f(allacc[row_lo, 11], 1),
                lane_f(allacc[row_hi, 11], 0), lane_f(allacc[row_hi, 11], 1))

    b_lo_f, below_lo, b_hi_f, below_hi = rank_search(low_f, high_f, 0, 1)
    b_lo = b_lo_f.astype(jnp.int32)
    b_hi = b_hi_f.astype(jnp.int32)

    zero_hist(HROWS * 8)

    @pl.when(wid == 0)
    def _():
        pltpu.sync_copy(hist_v, hist_sh)
    plsc.subcore_barrier()

    blo16 = splat_i(b_lo)
    bhi16 = splat_i(b_hi)

    def vecB(i, carry):
        minhi = carry
        xi = i * 48 + IOTA * 3
        yv = plsc.load_gather(xyzbuf, [xi + 1])
        opv = opbuf[pl.ds(i * 16, 16)]
        vis = opv > OPT
        kk = key_of(yv)
        top = lax.shift_right_logical(kk, C16)
        lo16 = lax.bitwise_and(kk, jnp.full((16,), 65535, jnp.int32))
        in_lo = vis & (top == blo16)
        plsc.addupdate_scatter(
            hist_v,
            [lax.shift_right_logical(lo16, C7), lax.bitwise_and(lo16, C127)],
            ONES16, mask=in_lo)
        in_hi = vis & (top == bhi16)
        minhi = jnp.minimum(minhi, jnp.where(in_hi, lo16, 65536))
        return minhi

    def chunkB(kk_, carry):
        dma_chunk(wid + kk_ * 16, False)
        return lax.fori_loop(0, NVEC, vecB, carry)

    minhi = lax.fori_loop(0, nchunks, chunkB, jnp.full((16,), 65536, jnp.int32))

    combine_hist(4)
    accbuf[0, :] = splat_f(jnp.min(minhi).astype(jnp.float32))
    pltpu.sync_copy(accbuf.at[0], acc_sh.at[wid, 9])
    plsc.subcore_barrier()

    v2_lo_f, _, v2_hi_same_f, _ = rank_search(
        low_f - below_lo, high_f - below_hi, 2, 3)

    minhi_g = jnp.float32(65536.0)
    for t in range(16):
        minhi_g = jnp.minimum(minhi_g, lane_f(allacc[t, 9], 0))
    same = b_lo == b_hi
    v2_hi_f = jnp.where(same, v2_hi_same_f, minhi_g)

    def inv_key(kint):
        kv = splat_i(kint)
        bits = jnp.where(kv < 0, lax.bitwise_xor(kv, MINV16),
                         lax.bitwise_not(kv))
        return lane_f(plsc.bitcast(bits, jnp.float32), 0)

    val_lo = inv_key(b_lo * 65536 + v2_lo_f.astype(jnp.int32))
    val_hi = inv_key(b_hi * 65536 + v2_hi_f.astype(jnp.int32))
    floor_h = val_lo * lw + val_hi * hw

    zero_hist(VROWS * 8)

    @pl.when(wid == 0)
    def _():
        pltpu.sync_copy(hist_v.at[pl.ds(0, VROWS)], hist_sh.at[pl.ds(0, VROWS)])
    plsc.subcore_barrier()

    ex_x = jnp.clip(mxx_g - mnx_g, 1e-4, None)
    ex_y = jnp.clip(mxy_g - mny_g, 1e-4, None)
    ex_z = jnp.clip(mxz_g - mnz_g, 1e-4, None)
    fl16 = splat_f(floor_h)
    mnx16, mny16, mnz16 = splat_f(mnx_g), splat_f(mny_g), splat_f(mnz_g)
    exx16, exy16, exz16 = splat_f(ex_x), splat_f(ex_y), splat_f(ex_z)

    def vox1(v, mn, ex):
        nv = (v - mn) / ex * jnp.float32(R - 1)
        return jnp.clip(nv.astype(jnp.int32), 0, R - 1)

    def vecC(i, carry):
        RU, LCS = carry
        xi = i * 48 + IOTA * 3
        xv = plsc.load_gather(xyzbuf, [xi])
        yv = plsc.load_gather(xyzbuf, [xi + 1])
        zv = plsc.load_gather(xyzbuf, [xi + 2])
        syv = plsc.load_gather(sclbuf, [xi + 1])
        opv = opbuf[pl.ds(i * 16, 16)]
        vis = opv > OPT
        w = jnp.where(vis, opv, 0.0)
        RU = RU + w * jnp.maximum(fl16 - yv, 0.0)
        LCS = LCS + w * jnp.maximum(fl16 - (yv - syv), 0.0)
        flat = (vox1(xv, mnx16, exx16) * 1024 + vox1(yv, mny16, exy16) * 32
                + vox1(zv, mnz16, exz16))
        plsc.addupdate_scatter(
            hist_v,
            [lax.shift_right_logical(flat, C7), lax.bitwise_and(flat, C127)],
            w, mask=vis)
        return (RU, LCS)

    def chunkC(kk_, carry):
        dma_chunk(wid + kk_ * 16, True)
        return lax.fori_loop(0, NVEC, vecC, carry)

    RU, LCS = lax.fori_loop(0, nchunks, chunkC, (ZERO16, ZERO16))

    combine_hist(2)
    accbuf[0, :] = RU
    accbuf[1, :] = LCS
    pltpu.sync_copy(accbuf.at[pl.ds(0, 2)], acc_sh.at[wid, pl.ds(9, 2)])
    plsc.subcore_barrier()

    @pl.when(wid == 0)
    def _():
        pltpu.sync_copy(acc_sh, allacc)
        RU_g, LCS_g = 0.0, 0.0
        for t in range(16):
            RU_g = RU_g + jnp.sum(allacc[t, 9])
            LCS_g = LCS_g + jnp.sum(allacc[t, 10])
        ov = jnp.where(IOTA == 0, splat_f(S_g), 0.0)
        ov = jnp.where(IOTA == 1, splat_f(Sh_g), ov)
        ov = jnp.where(IOTA == 2, splat_f(Sh2_g), ov)
        ov = jnp.where(IOTA == 3, splat_f(RU_g), ov)
        ov = jnp.where(IOTA == 4, splat_f(LCS_g), ov)
        outbuf[...] = ov
        pltpu.sync_copy(outbuf, scal_out)
        pltpu.sync_copy(hist_sh.at[pl.ds(0, VROWS)], grid_out)


@jax.jit
def _sc_call(xyzf, sclf, opf):
    mesh = plsc.VectorSubcoreMesh(
        core_axis_name="c", subcore_axis_name="s", num_cores=1)
    return pl.kernel(
        _sc_body,
        out_type=[
            jax.ShapeDtypeStruct((16,), jnp.float32),
            jax.ShapeDtypeStruct((VROWS, 128), jnp.float32),
        ],
        mesh=mesh,
        compiler_params=pltpu.CompilerParams(
            use_tc_tiling_on_sc=False, needs_layout_passes=False),
        scratch_types=[
            pltpu.VMEM((HROWS, 128), jnp.float32),
            pltpu.VMEM((12288,), jnp.float32),
            pltpu.VMEM((12288,), jnp.float32),
            pltpu.VMEM((4096,), jnp.float32),
            pltpu.VMEM((32, 128), jnp.float32),
            pltpu.VMEM((12, 16), jnp.float32),
            pltpu.VMEM((16,), jnp.float32),
            pltpu.VMEM((16,), jnp.float32),
            pltpu.VMEM((4, 128), jnp.int32),
            pltpu.VMEM((16, 12, 16), jnp.float32),
            pltpu.VMEM_SHARED((HROWS, 128), jnp.float32),
            pltpu.VMEM_SHARED((16, 12, 16), jnp.float32),
        ],
    )(xyzf, sclf, opf)


def _tc_body(scal_ref, grid_ref, out_ref):
    g = grid_ref[...]
    gn = g / (jnp.max(g) + jnp.float32(1e-8))
    st = jnp.float32(0.3)
    up0 = jnp.concatenate([gn[1:], gn[31:32]], axis=0)
    dn0 = jnp.concatenate([gn[0:1], gn[:31]], axis=0)
    up1 = jnp.concatenate([gn[:, 1:], gn[:, 31:32]], axis=1)
    dn1 = jnp.concatenate([gn[:, 0:1], gn[:, :31]], axis=1)
    up2 = jnp.concatenate([gn[:, :, 1:], gn[:, :, 31:32]], axis=2)
    dn2 = jnp.concatenate([gn[:, :, 0:1], gn[:, :, :31]], axis=2)
    interior = (up0 + dn0 + up1 + dn1 + up2 + dn2) / jnp.float32(6.0)
    surface = (gn > st).astype(jnp.float32)
    L_s = jnp.sum(jnp.maximum(st - interior, 0.0) * surface) / jnp.float32(R * R * R)
    S_g = scal_ref[0, 0]
    Sh_g = scal_ref[0, 1]
    Sh2_g = scal_ref[0, 2]
    RU_g = scal_ref[0, 3]
    LCS_g = scal_ref[0, 4]
    Sp = S_g + jnp.float32(1e-8)
    h_mean = Sh_g / Sp
    h_var = (Sh2_g - 2.0 * h_mean * Sh_g + h_mean * h_mean * S_g) / Sp
    L_g = h_var * jnp.float32(0.1) + RU_g / Sp
    L_c = LCS_g / jnp.float32(2000000)
    L_t = (jnp.float32(0.01) * L_g + jnp.float32(0.05) * L_c
           + jnp.float32(0.02) * L_s)
    out_ref[0] = L_t
    out_ref[1] = L_g
    out_ref[2] = L_c
    out_ref[3] = L_s


@jax.jit
def _tc_call(scal, grid3):
    return pl.pallas_call(
        _tc_body,
        out_shape=jax.ShapeDtypeStruct((4,), jnp.float32),
        in_specs=[
            pl.BlockSpec(memory_space=pltpu.SMEM),
            pl.BlockSpec(memory_space=pltpu.VMEM),
        ],
        out_specs=pl.BlockSpec(memory_space=pltpu.SMEM),
    )(scal, grid3)


def kernel(xyz, scaling, opacity):
    scal16, grid = _sc_call(
        xyz.reshape(-1), scaling.reshape(-1), opacity.reshape(-1))
    out4 = _tc_call(scal16.reshape(1, 16), grid.reshape(R, R, R))
    return out4[0], out4[1], out4[2], out4[3]


# speedup vs baseline: 27.0947x; 27.0947x over previous
import functools

import jax
import jax.numpy as jnp
from jax import lax
from jax.experimental import pallas as pl
from jax.experimental.pallas import tpu as pltpu
from jax.experimental.pallas import tpu_sc as plsc

N = 2_000_000
C = 4_000
NCHUNK = N // C
NVEC = C // 16
HROWS = 512
VROWS = 256
OPT = 0.05
R = 32

_I32MIN = -2147483648


def _sc_body(x_hbm, y_hbm, z_hbm, sy_hbm, op_hbm, scal_out, grid_out,
             hist_v, xbuf, ybuf, zbuf, sybuf, opbuf, histslice, accbuf,
             resbuf, outbuf, idxrows, allacc, hist_sh, acc_sh, sem0, sem1):
    wid = lax.axis_index("s")
    IOTA = lax.broadcasted_iota(jnp.int32, (16,), 0)
    ZERO16 = jnp.zeros((16,), jnp.float32)
    ONES16 = jnp.full((16,), 1.0, jnp.float32)
    INF16 = jnp.full((16,), jnp.inf, jnp.float32)
    C16 = jnp.full((16,), 16, jnp.int32)
    C31 = jnp.full((16,), 31, jnp.int32)
    C7 = jnp.full((16,), 7, jnp.int32)
    C127 = jnp.full((16,), 127, jnp.int32)
    MINV16 = jnp.full((16,), _I32MIN, jnp.int32)

    def lane_f(v, k):
        return jnp.sum(jnp.where(IOTA == k, v, 0.0))

    def splat_f(s):
        return jnp.full((16,), s, jnp.float32)

    def splat_i(s):
        return jnp.full((16,), s, jnp.int32)

    def zero_hist(nvecs):
        @plsc.parallel_loop(0, nvecs, unroll=8)
        def _(i):
            r = lax.shift_right_logical(i, 3)
            c = lax.bitwise_and(i, 7) * 16
            hist_v[r, pl.ds(c, 16)] = ZERO16

    def combine_hist(nchunks128):
        for k in range(nchunks128):
            pltpu.sync_copy(hist_v.at[pl.ds(k * 128, 128)],
                            hist_sh.at[idxrows.at[k]], add=True)

    def key_of(yv):
        b = plsc.bitcast(yv, jnp.int32)
        m = lax.shift_right_arithmetic(b, C31)
        return lax.bitwise_xor(b, lax.bitwise_or(m, MINV16))

    BUFS = {"A": (xbuf, ybuf, zbuf, opbuf), "B": (ybuf, opbuf),
            "C": (xbuf, ybuf, zbuf, sybuf, opbuf)}
    HBMS = {"A": (x_hbm, y_hbm, z_hbm, op_hbm), "B": (y_hbm, op_hbm),
            "C": (x_hbm, y_hbm, z_hbm, sy_hbm, op_hbm)}

    def start_dma(cid, slot, sem, phase):
        s1 = pl.multiple_of(cid * C, 8)
        for hbm, buf in zip(HBMS[phase], BUFS[phase]):
            pltpu.async_copy(hbm.at[pl.ds(s1, C)],
                             buf.at[pl.ds(slot * 4096, C)], sem)

    def wait_dma(slot, sem, phase):
        for hbm, buf in zip(HBMS[phase], BUFS[phase]):
            pltpu.make_async_copy(hbm.at[pl.ds(0, C)],
                                  buf.at[pl.ds(slot * 4096, C)], sem).wait()

    def phase_loop(phase, vec_fn, init):
        start_dma(wid, 0, sem0, phase)

        def half(slot, carry, valid):
            base = slot * 4096

            @plsc.parallel_loop(0, NVEC, unroll=4, carry=carry)
            def done(i, c):
                return vec_fn(base + i * 16, c, valid)
            return done

        def pair(kk, carry):
            cid0 = wid + (2 * kk) * 16
            cid1 = cid0 + 16

            @pl.when(cid1 < NCHUNK)
            def _():
                start_dma(cid1, 1, sem1, phase)
            wait_dma(0, sem0, phase)
            carry = half(0, carry, cid0 < NCHUNK)

            @pl.when(kk < 15)
            def _():
                start_dma(cid0 + 32, 0, sem0, phase)

            @pl.when(cid1 < NCHUNK)
            def _():
                wait_dma(1, sem1, phase)
            carry = half(1, carry, cid1 < NCHUNK)
            return carry
        return lax.fori_loop(0, 16, pair, init)

    for k in range(4):
        for j in range(8):
            idxrows[k, pl.ds(j * 16, 16)] = IOTA + (k * 128 + j * 16)

    zero_hist(HROWS * 8)

    @pl.when(wid == 0)
    def _():
        pltpu.sync_copy(hist_v, hist_sh)
    plsc.subcore_barrier()

    def vecA(o, carry, valid):
        (mnx, mxx, mny, mxy, mnz, mxz, cnt, S, Sh, Sh2) = carry
        xv = xbuf[pl.ds(o, 16)]
        yv = ybuf[pl.ds(o, 16)]
        zv = zbuf[pl.ds(o, 16)]
        opv = opbuf[pl.ds(o, 16)]
        vis = (opv > OPT) & valid
        w = jnp.where(vis, opv, 0.0)
        mnx = jnp.minimum(mnx, jnp.where(vis, xv, INF16))
        mxx = jnp.maximum(mxx, jnp.where(vis, xv, -INF16))
        mny = jnp.minimum(mny, jnp.where(vis, yv, INF16))
        mxy = jnp.maximum(mxy, jnp.where(vis, yv, -INF16))
        mnz = jnp.minimum(mnz, jnp.where(vis, zv, INF16))
        mxz = jnp.maximum(mxz, jnp.where(vis, zv, -INF16))
        cnt = cnt + jnp.where(vis, 1.0, 0.0)
        S = S + w
        Sh = Sh + w * yv
        Sh2 = Sh2 + w * yv * yv
        top = lax.shift_right_logical(key_of(yv), C16)
        plsc.addupdate_scatter(
            hist_v,
            [lax.shift_right_logical(top, C7), lax.bitwise_and(top, C127)],
            ONES16, mask=vis)
        return (mnx, mxx, mny, mxy, mnz, mxz, cnt, S, Sh, Sh2)

    accA = phase_loop("A", vecA, (
        INF16, -INF16, INF16, -INF16, INF16, -INF16,
        ZERO16, ZERO16, ZERO16, ZERO16))

    combine_hist(4)
    for j in range(10):
        accbuf[j, :] = accA[j]
    pltpu.sync_copy(accbuf, acc_sh.at[wid])
    plsc.subcore_barrier()
    pltpu.sync_copy(acc_sh, allacc)

    mnx_g, mxx_g = jnp.inf, -jnp.inf
    mny_g, mxy_g = jnp.inf, -jnp.inf
    mnz_g, mxz_g = jnp.inf, -jnp.inf
    n_g, S_g, Sh_g, Sh2_g = 0.0, 0.0, 0.0, 0.0
    for t in range(16):
        mnx_g = jnp.minimum(mnx_g, jnp.min(allacc[t, 0]))
        mxx_g = jnp.maximum(mxx_g, jnp.max(allacc[t, 1]))
        mny_g = jnp.minimum(mny_g, jnp.min(allacc[t, 2]))
        mxy_g = jnp.maximum(mxy_g, jnp.max(allacc[t, 3]))
        mnz_g = jnp.minimum(mnz_g, jnp.min(allacc[t, 4]))
        mxz_g = jnp.maximum(mxz_g, jnp.max(allacc[t, 5]))
        n_g = n_g + jnp.sum(allacc[t, 6])
        S_g = S_g + jnp.sum(allacc[t, 7])
        Sh_g = Sh_g + jnp.sum(allacc[t, 8])
        Sh2_g = Sh2_g + jnp.sum(allacc[t, 9])

    pos = jnp.float32(OPT) * (n_g - 1.0)
    low_i = pos.astype(jnp.int32)
    low_f = low_i.astype(jnp.float32)
    high_f = jnp.where(pos > low_f, low_f + 1.0, low_f)
    hw = pos - low_f
    lw = 1.0 - hw

    def slice_totals():
        pltpu.sync_copy(hist_sh.at[pl.ds(wid * 32, 32)], histslice)

        def b(p, a):
            r = lax.shift_right_logical(p, 3)
            c = lax.bitwise_and(p, 7) * 16
            return a + histslice[r, pl.ds(c, 16)]
        v = lax.fori_loop(0, 256, b, ZERO16)
        accbuf[0, :] = splat_f(jnp.sum(v))
        pltpu.sync_copy(accbuf.at[0], acc_sh.at[wid, 10])

    def find_owner(rank_f):
        cum, owner, below = 0.0, jnp.int32(-1), 0.0
        for t in range(16):
            tot = lane_f(allacc[t, 10], 0)
            ncum = cum + tot
            hit = (ncum >= rank_f + 1.0) & (owner < 0)
            owner = jnp.where(hit, t, owner)
            below = jnp.where(hit, cum, below)
            cum = ncum
        return owner, below

    def owner_scan(rank_f, below_slice, res_row):
        def b(p, car):
            cum, fb, fbel = car
            r = lax.shift_right_logical(p, 3)
            c = lax.bitwise_and(p, 7) * 16
            vec = histslice[r, pl.ds(c, 16)]
            ic = plsc.cumsum(vec) + cum
            hit = ic >= rank_f + 1.0
            any_hit = jnp.any(hit)
            f = jnp.min(plsc.all_reduce_ffs(hit))
            binv = jnp.sum(jnp.where(IOTA == f, vec, 0.0))
            icf = jnp.sum(jnp.where(IOTA == f, ic, 0.0))
            gb = (wid * 4096 + p * 16 + f).astype(jnp.float32)
            upd = any_hit & (fb < 0.0)
            fb = jnp.where(upd, gb, fb)
            fbel = jnp.where(upd, icf - binv, fbel)
            return (cum + jnp.sum(vec), fb, fbel)
        _, fb, fbel = lax.fori_loop(0, 256, b, (below_slice, -1.0, 0.0))
        resbuf[...] = jnp.where(IOTA == 0, splat_f(fb), splat_f(fbel))
        pltpu.sync_copy(resbuf, acc_sh.at[res_row, 11])

    def rank_search(rank_lo_f, rank_hi_f, row_lo, row_hi):
        slice_totals()
        plsc.subcore_barrier()
        pltpu.sync_copy(acc_sh, allacc)
        own_lo, bel_lo = find_owner(rank_lo_f)
        own_hi, bel_hi = find_owner(rank_hi_f)

        @pl.when(wid == own_lo)
        def _():
            owner_scan(rank_lo_f, bel_lo, row_lo)

        @pl.when(wid == own_hi)
        def _():
            owner_scan(rank_hi_f, bel_hi, row_hi)
        plsc.subcore_barrier()
        pltpu.sync_copy(acc_sh, allacc)
        return (lane_f(allacc[row_lo, 11], 0), lane_f(allacc[row_lo, 11], 1),
                lane_f(allacc[row_hi, 11], 0), lane_f(allacc[row_hi, 11], 1))

    b_lo_f, below_lo, b_hi_f, below_hi = rank_search(low_f, high_f, 0, 1)
    b_lo = b_lo_f.astype(jnp.int32)
    b_hi = b_hi_f.astype(jnp.int32)

    zero_hist(HROWS * 8)

    @pl.when(wid == 0)
    def _():
        pltpu.sync_copy(hist_v, hist_sh)
    plsc.subcore_barrier()

    blo16 = splat_i(b_lo)
    bhi16 = splat_i(b_hi)

    def vecB(o, carry, valid):
        minhi = carry
        yv = ybuf[pl.ds(o, 16)]
        opv = opbuf[pl.ds(o, 16)]
        vis = (opv > OPT) & valid
        kk = key_of(yv)
        top = lax.shift_right_logical(kk, C16)
        lo16 = lax.bitwise_and(kk, jnp.full((16,), 65535, jnp.int32))
        in_lo = vis & (top == blo16)
        plsc.addupdate_scatter(
            hist_v,
            [lax.shift_right_logical(lo16, C7), lax.bitwise_and(lo16, C127)],
            ONES16, mask=in_lo)
        in_hi = vis & (top == bhi16)
        minhi = jnp.minimum(minhi, jnp.where(in_hi, lo16, 65536))
        return minhi

    minhi = phase_loop("B", vecB, jnp.full((16,), 65536, jnp.int32))

    combine_hist(4)
    accbuf[0, :] = splat_f(jnp.min(minhi).astype(jnp.float32))
    pltpu.sync_copy(accbuf.at[0], acc_sh.at[wid, 9])
    plsc.subcore_barrier()

    v2_lo_f, _, v2_hi_same_f, _ = rank_search(
        low_f - below_lo, high_f - below_hi, 2, 3)

    minhi_g = jnp.float32(65536.0)
    for t in range(16):
        minhi_g = jnp.minimum(minhi_g, lane_f(allacc[t, 9], 0))
    same = b_lo == b_hi
    v2_hi_f = jnp.where(same, v2_hi_same_f, minhi_g)

    def inv_key(kint):
        kv = splat_i(kint)
        bits = jnp.where(kv < 0, lax.bitwise_xor(kv, MINV16),
                         lax.bitwise_not(kv))
        return lane_f(plsc.bitcast(bits, jnp.float32), 0)

    val_lo = inv_key(b_lo * 65536 + v2_lo_f.astype(jnp.int32))
    val_hi = inv_key(b_hi * 65536 + v2_hi_f.astype(jnp.int32))
    floor_h = val_lo * lw + val_hi * hw

    zero_hist(VROWS * 8)

    @pl.when(wid == 0)
    def _():
        pltpu.sync_copy(hist_v.at[pl.ds(0, VROWS)], hist_sh.at[pl.ds(0, VROWS)])
    plsc.subcore_barrier()

    ex_x = jnp.clip(mxx_g - mnx_g, 1e-4, None)
    ex_y = jnp.clip(mxy_g - mny_g, 1e-4, None)
    ex_z = jnp.clip(mxz_g - mnz_g, 1e-4, None)
    fl16 = splat_f(floor_h)
    mnx16, mny16, mnz16 = splat_f(mnx_g), splat_f(mny_g), splat_f(mnz_g)
    exx16, exy16, exz16 = splat_f(ex_x), splat_f(ex_y), splat_f(ex_z)

    def vox1(v, mn, ex):
        nv = (v - mn) / ex * jnp.float32(R - 1)
        return jnp.clip(nv.astype(jnp.int32), 0, R - 1)

    def vecC(o, carry, valid):
        RU, LCS = carry
        xv = xbuf[pl.ds(o, 16)]
        yv = ybuf[pl.ds(o, 16)]
        zv = zbuf[pl.ds(o, 16)]
        syv = sybuf[pl.ds(o, 16)]
        opv = opbuf[pl.ds(o, 16)]
        vis = (opv > OPT) & valid
        w = jnp.where(vis, opv, 0.0)
        RU = RU + w * jnp.maximum(fl16 - yv, 0.0)
        LCS = LCS + w * jnp.maximum(fl16 - (yv - syv), 0.0)
        flat = (vox1(xv, mnx16, exx16) * 1024 + vox1(yv, mny16, exy16) * 32
                + vox1(zv, mnz16, exz16))
        plsc.addupdate_scatter(
            hist_v,
            [lax.shift_right_logical(flat, C7), lax.bitwise_and(flat, C127)],
            w, mask=vis)
        return (RU, LCS)

    RU, LCS = phase_loop("C", vecC, (ZERO16, ZERO16))

    combine_hist(2)
    accbuf[0, :] = RU
    accbuf[1, :] = LCS
    pltpu.sync_copy(accbuf.at[pl.ds(0, 2)], acc_sh.at[wid, pl.ds(9, 2)])
    plsc.subcore_barrier()

    @pl.when(wid == 0)
    def _():
        pltpu.sync_copy(acc_sh, allacc)
        RU_g, LCS_g = 0.0, 0.0
        for t in range(16):
            RU_g = RU_g + jnp.sum(allacc[t, 9])
            LCS_g = LCS_g + jnp.sum(allacc[t, 10])
        ov = jnp.where(IOTA == 0, splat_f(S_g), 0.0)
        ov = jnp.where(IOTA == 1, splat_f(Sh_g), ov)
        ov = jnp.where(IOTA == 2, splat_f(Sh2_g), ov)
        ov = jnp.where(IOTA == 3, splat_f(RU_g), ov)
        ov = jnp.where(IOTA == 4, splat_f(LCS_g), ov)
        outbuf[...] = ov
        pltpu.sync_copy(outbuf, scal_out)
        pltpu.sync_copy(hist_sh.at[pl.ds(0, VROWS)], grid_out)


@jax.jit
def _sc_call(x, y, z, sy, opf):
    mesh = plsc.VectorSubcoreMesh(
        core_axis_name="c", subcore_axis_name="s", num_cores=1)
    return pl.kernel(
        _sc_body,
        out_type=[
            jax.ShapeDtypeStruct((16,), jnp.float32),
            jax.ShapeDtypeStruct((VROWS, 128), jnp.float32),
        ],
        mesh=mesh,
        compiler_params=pltpu.CompilerParams(
            use_tc_tiling_on_sc=False, needs_layout_passes=False),
        scratch_types=[
            pltpu.VMEM((HROWS, 128), jnp.float32),
            pltpu.VMEM((8192,), jnp.float32),
            pltpu.VMEM((8192,), jnp.float32),
            pltpu.VMEM((8192,), jnp.float32),
            pltpu.VMEM((8192,), jnp.float32),
            pltpu.VMEM((8192,), jnp.float32),
            pltpu.VMEM((32, 128), jnp.float32),
            pltpu.VMEM((12, 16), jnp.float32),
            pltpu.VMEM((16,), jnp.float32),
            pltpu.VMEM((16,), jnp.float32),
            pltpu.VMEM((4, 128), jnp.int32),
            pltpu.VMEM((16, 12, 16), jnp.float32),
            pltpu.VMEM_SHARED((HROWS, 128), jnp.float32),
            pltpu.VMEM_SHARED((16, 12, 16), jnp.float32),
            pltpu.SemaphoreType.DMA,
            pltpu.SemaphoreType.DMA,
        ],
    )(x, y, z, sy, opf)


def _tc_body(scal_ref, grid_ref, out_ref):
    g = grid_ref[...]
    gn = g / (jnp.max(g) + jnp.float32(1e-8))
    st = jnp.float32(0.3)
    up0 = jnp.concatenate([gn[1:], gn[31:32]], axis=0)
    dn0 = jnp.concatenate([gn[0:1], gn[:31]], axis=0)
    up1 = jnp.concatenate([gn[:, 1:], gn[:, 31:32]], axis=1)
    dn1 = jnp.concatenate([gn[:, 0:1], gn[:, :31]], axis=1)
    up2 = jnp.concatenate([gn[:, :, 1:], gn[:, :, 31:32]], axis=2)
    dn2 = jnp.concatenate([gn[:, :, 0:1], gn[:, :, :31]], axis=2)
    interior = (up0 + dn0 + up1 + dn1 + up2 + dn2) / jnp.float32(6.0)
    surface = (gn > st).astype(jnp.float32)
    L_s = jnp.sum(jnp.maximum(st - interior, 0.0) * surface) / jnp.float32(R * R * R)
    S_g = scal_ref[0, 0]
    Sh_g = scal_ref[0, 1]
    Sh2_g = scal_ref[0, 2]
    RU_g = scal_ref[0, 3]
    LCS_g = scal_ref[0, 4]
    Sp = S_g + jnp.float32(1e-8)
    h_mean = Sh_g / Sp
    h_var = (Sh2_g - 2.0 * h_mean * Sh_g + h_mean * h_mean * S_g) / Sp
    L_g = h_var * jnp.float32(0.1) + RU_g / Sp
    L_c = LCS_g / jnp.float32(2000000)
    L_t = (jnp.float32(0.01) * L_g + jnp.float32(0.05) * L_c
           + jnp.float32(0.02) * L_s)
    out_ref[0] = L_t
    out_ref[1] = L_g
    out_ref[2] = L_c
    out_ref[3] = L_s


@jax.jit
def _tc_call(scal, grid3):
    return pl.pallas_call(
        _tc_body,
        out_shape=jax.ShapeDtypeStruct((4,), jnp.float32),
        in_specs=[
            pl.BlockSpec(memory_space=pltpu.SMEM),
            pl.BlockSpec(memory_space=pltpu.VMEM),
        ],
        out_specs=pl.BlockSpec(memory_space=pltpu.SMEM),
    )(scal, grid3)


def kernel(xyz, scaling, opacity):
    scal16, grid = _sc_call(
        xyz[:, 0], xyz[:, 1], xyz[:, 2], scaling[:, 1],
        opacity.reshape(-1))
    out4 = _tc_call(scal16.reshape(1, 16), grid.reshape(R, R, R))
    return out4[0], out4[1], out4[2], out4[3]
